# Initial kernel scaffold; baseline (speedup 1.0000x reference)
import jax
import jax.numpy as jnp
from jax.experimental import pallas as pl


def kernel(x, edge_index, edge_attr, W_gat, att_src, att_dst, W_edge, att_edge, bias, gamma, beta):
    raise NotImplementedError("write your pallas kernel here")


# speedup vs baseline: 42.8262x; 42.8262x over previous
import functools

import jax
import jax.numpy as jnp
from jax import lax
from jax.experimental import pallas as pl
from jax.experimental.pallas import tpu as pltpu
from jax.experimental.pallas import tpu_sc as plsc

N = 10000
E = 320000
D = 128
HP = 16
CHUNK = 128
NC = 2
NS = 16
NW = NC * NS
CHUNKS_PER_W = 79
TOTAL_CHUNKS = NW * CHUNKS_PER_W
E_PAD = TOTAL_CHUNKS * CHUNK
NPAD = 10240
ROWS_PER_TILE = NPAD // NS
NBLK = 400
R32 = E // 32
RPAD = E_PAD // 32
EBLK = 400

_GDN = lax.GatherDimensionNumbers(
    offset_dims=(), collapsed_slice_dims=(0,), start_index_map=(0,))


def _splat(vec, h):
    idx = jnp.full((16, 1), h, jnp.int32)
    return lax.gather(vec, idx, _GDN, (1,),
                      mode=lax.GatherScatterMode.PROMISE_IN_BOUNDS)


def _sel8(rows, cols):
    k = lax.broadcasted_iota(jnp.int32, (rows, cols), 0) // 16
    h = lax.broadcasted_iota(jnp.int32, (rows, cols), 1)
    return ((k == h) & (h < 8)).astype(jnp.float32)


def _pre_body(x_ref, wg_ref, asf_ref, adf_ref, xp_ref, asrc_ref, adst_ref):
    xp = jnp.dot(x_ref[...], wg_ref[...], preferred_element_type=jnp.float32)
    xp_ref[...] = xp
    sel = _sel8(D, HP)
    asrc_ref[...] = jnp.dot(xp * asf_ref[...], sel,
                            preferred_element_type=jnp.float32)
    adst_ref[...] = jnp.dot(xp * adf_ref[...], sel,
                            preferred_element_type=jnp.float32)


def _tc_pre(x, wg, asf, adf):
    return pl.pallas_call(
        _pre_body,
        grid=(N // NBLK,),
        in_specs=[
            pl.BlockSpec((NBLK, D), lambda i: (i, 0)),
            pl.BlockSpec((D, D), lambda i: (0, 0)),
            pl.BlockSpec((1, D), lambda i: (0, 0)),
            pl.BlockSpec((1, D), lambda i: (0, 0)),
        ],
        out_specs=[
            pl.BlockSpec((NBLK, D), lambda i: (i, 0)),
            pl.BlockSpec((NBLK, HP), lambda i: (i, 0)),
            pl.BlockSpec((NBLK, HP), lambda i: (i, 0)),
        ],
        out_shape=[
            jax.ShapeDtypeStruct((NPAD, D), jnp.float32),
            jax.ShapeDtypeStruct((NPAD, HP), jnp.float32),
            jax.ShapeDtypeStruct((NPAD, HP), jnp.float32),
        ],
    )(x, wg, asf, adf)


def _edge_body(eattr_ref, we_ref, aef_ref, o0_ref, o1_ref, o2_ref, o3_ref):
    m = jnp.dot(we_ref[...] * aef_ref[...], _sel8(D, HP),
                preferred_element_type=jnp.float32)
    p2 = (lax.broadcasted_iota(jnp.int32, (HP, D), 1) % HP
          == lax.broadcasted_iota(jnp.int32, (HP, D), 0)).astype(jnp.float32)
    blk = ((lax.broadcasted_iota(jnp.int32, (D, D), 0) % 32) // 4
           == lax.broadcasted_iota(jnp.int32, (D, D), 1) // HP)
    blkf = blk.astype(jnp.float32)
    a_ri = lax.broadcasted_iota(jnp.int32, (D, 4), 0)
    a_c = lax.broadcasted_iota(jnp.int32, (D, 4), 1)
    xin = eattr_ref[...]
    outs = (o0_ref, o1_ref, o2_ref, o3_ref)
    for q in range(4):
        aq = ((a_ri % 4 == a_c) & (a_ri // 32 == q)).astype(jnp.float32)
        kq = jnp.dot(jnp.dot(aq, m, preferred_element_type=jnp.float32), p2,
                     preferred_element_type=jnp.float32) * blkf
        outs[q][...] = jnp.dot(xin, kq, preferred_element_type=jnp.float32)


def _tc_edge(eattr2d, we, aef):
    return pl.pallas_call(
        _edge_body,
        grid=(R32 // EBLK,),
        in_specs=[
            pl.BlockSpec((EBLK, D), lambda i: (i, 0)),
            pl.BlockSpec((4, D), lambda i: (0, 0)),
            pl.BlockSpec((1, D), lambda i: (0, 0)),
        ],
        out_specs=[pl.BlockSpec((EBLK, D), lambda i: (i, 0))] * 4,
        out_shape=[jax.ShapeDtypeStruct((RPAD, D), jnp.float32)] * 4,
    )(eattr2d, we, aef)


def _post_body(acc_ref, ss_ref, x_ref, b_ref, g_ref, be_ref, out_ref):
    acc = acc_ref[0] + acc_ref[1]
    ss = ss_ref[0] + ss_ref[1]
    expand = _sel8(D, HP).T
    den = jnp.dot(ss, expand, preferred_element_type=jnp.float32) + 1e-16
    out = acc / den + b_ref[...] + x_ref[...]
    mu = jnp.mean(out, axis=-1, keepdims=True)
    var = jnp.mean((out - mu) * (out - mu), axis=-1, keepdims=True)
    y = (out - mu) * lax.rsqrt(var + 1e-5) * g_ref[...] + be_ref[...]
    out_ref[...] = jnp.where(y > 0, y, jnp.exp(y) - 1.0)


def _tc_post(acc, ss, x, b, g, be):
    return pl.pallas_call(
        _post_body,
        grid=(N // 1000,),
        in_specs=[
            pl.BlockSpec((NC, 1000, D), lambda i: (0, i, 0)),
            pl.BlockSpec((NC, 1000, HP), lambda i: (0, i, 0)),
            pl.BlockSpec((1000, D), lambda i: (i, 0)),
            pl.BlockSpec((1, D), lambda i: (0, 0)),
            pl.BlockSpec((1, D), lambda i: (0, 0)),
            pl.BlockSpec((1, D), lambda i: (0, 0)),
        ],
        out_specs=pl.BlockSpec((1000, D), lambda i: (i, 0)),
        out_shape=jax.ShapeDtypeStruct((N, D), jnp.float32),
    )(acc, ss, x, b, g, be)


@functools.partial(
    pl.kernel,
    out_type=[
        jax.ShapeDtypeStruct((NC * NPAD, D), jnp.float32),
        jax.ShapeDtypeStruct((NC * NPAD, HP), jnp.float32),
    ],
    mesh=plsc.VectorSubcoreMesh(core_axis_name="c", subcore_axis_name="s"),
    compiler_params=pltpu.CompilerParams(use_tc_tiling_on_sc=False),
    scratch_types=[
        pltpu.VMEM((CHUNK,), jnp.int32),
        pltpu.VMEM((CHUNK,), jnp.int32),
        pltpu.VMEM((CHUNK // 8, D), jnp.float32),
        pltpu.VMEM((CHUNK, HP), jnp.float32),
        pltpu.VMEM((CHUNK, HP), jnp.float32),
        pltpu.VMEM((CHUNK, HP), jnp.float32),
        pltpu.VMEM((CHUNK, D), jnp.float32),
        pltpu.VMEM_SHARED((NPAD, D), jnp.float32),
        pltpu.VMEM_SHARED((NPAD, HP), jnp.float32),
        pltpu.SemaphoreType.DMA,
        pltpu.SemaphoreType.DMA,
        pltpu.SemaphoreType.DMA,
    ],
)
def _sc_edge(xp_hbm, asrc_hbm, adst_hbm, o0_hbm, o1_hbm, o2_hbm, o3_hbm,
             src_hbm, dst_hbm,
             acc_out, ss_out,
             src_v, dst_v, ae_v, ea_v, as_v, ad_v, x_v,
             acc_sh, ss_sh, sem1, sem2, sem3):
    cid = lax.axis_index("c")
    sid = lax.axis_index("s")
    wid = sid * NC + cid
    row0 = sid * ROWS_PER_TILE

    z16 = jnp.zeros((16,), jnp.float32)

    def _zrow(r, _):
        for j in range(D // 16):
            x_v[r, pl.ds(j * 16, 16)] = z16
        ea_v[r, :] = z16
        return 0

    lax.fori_loop(0, CHUNK, _zrow, 0)
    for k in range(ROWS_PER_TILE // CHUNK):
        pltpu.sync_copy(x_v, acc_sh.at[pl.ds(row0 + k * CHUNK, CHUNK), :])
        pltpu.sync_copy(ea_v, ss_sh.at[pl.ds(row0 + k * CHUNK, CHUNK), :])
    plsc.subcore_barrier()

    def _chunk(t, _):
        c = wid * CHUNKS_PER_W + t
        base = c * CHUNK
        pltpu.sync_copy(src_hbm.at[pl.ds(base, CHUNK)], src_v)
        pltpu.sync_copy(dst_hbm.at[pl.ds(base, CHUNK)], dst_v)
        cp1 = pltpu.async_copy(asrc_hbm.at[src_v], as_v, sem1)
        cp2 = pltpu.async_copy(adst_hbm.at[dst_v], ad_v, sem2)
        cp3 = pltpu.async_copy(xp_hbm.at[src_v], x_v, sem3)
        pltpu.sync_copy(o0_hbm.at[pl.ds(c * 4, 4), :], ae_v.at[0:4, :])
        pltpu.sync_copy(o1_hbm.at[pl.ds(c * 4, 4), :], ae_v.at[4:8, :])
        pltpu.sync_copy(o2_hbm.at[pl.ds(c * 4, 4), :], ae_v.at[8:12, :])
        pltpu.sync_copy(o3_hbm.at[pl.ds(c * 4, 4), :], ae_v.at[12:16, :])
        cp1.wait()
        cp2.wait()

        def _erow(r, _):
            base = 32 * (r % 4) + 8 * (r // 4)
            for a in range(8):
                e = base + a
                v = (as_v[e, :] + ad_v[e, :] + ae_v[r, pl.ds(a * HP, 16)])
                ea_v[e, :] = jnp.exp(jnp.maximum(v, 0.2 * v))
            return 0

        lax.fori_loop(0, CHUNK // 8, _erow, 0)
        cp3.wait()

        def _mrow(e, _):
            ea = ea_v[e, :]
            for h in range(8):
                w = _splat(ea, h)
                x_v[e, pl.ds(h * 16, 16)] = x_v[e, pl.ds(h * 16, 16)] * w
            return 0

        lax.fori_loop(0, CHUNK, _mrow, 0)
        pltpu.sync_copy(x_v, acc_sh.at[dst_v], add=True)
        pltpu.sync_copy(ea_v, ss_sh.at[dst_v], add=True)
        return 0

    lax.fori_loop(0, CHUNKS_PER_W, _chunk, 0)
    plsc.subcore_barrier()
    pltpu.sync_copy(acc_sh.at[pl.ds(row0, ROWS_PER_TILE), :],
                    acc_out.at[pl.ds(cid * NPAD + row0, ROWS_PER_TILE), :])
    pltpu.sync_copy(ss_sh.at[pl.ds(row0, ROWS_PER_TILE), :],
                    ss_out.at[pl.ds(cid * NPAD + row0, ROWS_PER_TILE), :])


def kernel(x, edge_index, edge_attr, W_gat, att_src, att_dst, W_edge,
           att_edge, bias, gamma, beta):
    src = edge_index[0].astype(jnp.int32)
    dst = edge_index[1].astype(jnp.int32)
    pad_e = E_PAD - E
    srcp = jnp.concatenate([src, jnp.full((pad_e,), N, jnp.int32)])
    dstp = jnp.concatenate([dst, jnp.full((pad_e,), N, jnp.int32)])
    asf = att_src.reshape(1, D)
    adf = att_dst.reshape(1, D)
    aef = att_edge.reshape(1, D)
    xp, asrc, adst = _tc_pre(x, W_gat, asf, adf)
    o0, o1, o2, o3 = _tc_edge(edge_attr.reshape(R32, D), W_edge, aef)
    acc, ss = _sc_edge(xp, asrc, adst, o0, o1, o2, o3, srcp, dstp)
    return _tc_post(acc.reshape(NC, NPAD, D), ss.reshape(NC, NPAD, HP),
                    x, bias.reshape(1, D), gamma.reshape(1, D),
                    beta.reshape(1, D))


# speedup vs baseline: 45.2799x; 1.0573x over previous
import functools

import jax
import jax.numpy as jnp
from jax import lax
from jax.experimental import pallas as pl
from jax.experimental.pallas import tpu as pltpu
from jax.experimental.pallas import tpu_sc as plsc

N = 10000
E = 320000
D = 128
HP = 16
CHUNK = 128
NC = 2
NS = 16
NW = NC * NS
CHUNKS_PER_W = 79
TOTAL_CHUNKS = NW * CHUNKS_PER_W
E_PAD = TOTAL_CHUNKS * CHUNK
NPAD = 10240
ROWS_PER_TILE = NPAD // NS
NBLK = 640
EBLK = 1264

_GDN = lax.GatherDimensionNumbers(
    offset_dims=(), collapsed_slice_dims=(0,), start_index_map=(0,))


def _splat(vec, h):
    idx = jnp.full((16, 1), h, jnp.int32)
    return lax.gather(vec, idx, _GDN, (1,),
                      mode=lax.GatherScatterMode.PROMISE_IN_BOUNDS)


def _sel8(rows, cols):
    k = lax.broadcasted_iota(jnp.int32, (rows, cols), 0) // 16
    h = lax.broadcasted_iota(jnp.int32, (rows, cols), 1)
    return ((k == h) & (h < 8)).astype(jnp.float32)


def _pre_body(x_ref, wg_ref, asf_ref, adf_ref, xp_ref, asrc_ref, adst_ref):
    xp = jnp.dot(x_ref[...], wg_ref[...], preferred_element_type=jnp.float32)
    xp_ref[...] = xp
    sel = _sel8(D, HP)
    asrc_ref[...] = jnp.dot(xp * asf_ref[...], sel,
                            preferred_element_type=jnp.float32)
    adst_ref[...] = jnp.dot(xp * adf_ref[...], sel,
                            preferred_element_type=jnp.float32)


def _tc_pre(x, wg, asf, adf):
    return pl.pallas_call(
        _pre_body,
        grid=(NPAD // NBLK,),
        in_specs=[
            pl.BlockSpec((NBLK, D), lambda i: (i, 0)),
            pl.BlockSpec((D, D), lambda i: (0, 0)),
            pl.BlockSpec((1, D), lambda i: (0, 0)),
            pl.BlockSpec((1, D), lambda i: (0, 0)),
        ],
        out_specs=[
            pl.BlockSpec((NBLK, D), lambda i: (i, 0)),
            pl.BlockSpec((NBLK, HP), lambda i: (i, 0)),
            pl.BlockSpec((NBLK, HP), lambda i: (i, 0)),
        ],
        out_shape=[
            jax.ShapeDtypeStruct((NPAD, D), jnp.float32),
            jax.ShapeDtypeStruct((NPAD, HP), jnp.float32),
            jax.ShapeDtypeStruct((NPAD, HP), jnp.float32),
        ],
    )(x, wg, asf, adf)


def _edge_body(eattr_ref, we_ref, aef_ref, out_ref):
    m = jnp.dot(we_ref[...] * aef_ref[...], _sel8(D, HP),
                preferred_element_type=jnp.float32)
    p1 = (lax.broadcasted_iota(jnp.int32, (32, 4), 0) % 4
          == lax.broadcasted_iota(jnp.int32, (32, 4), 1)).astype(jnp.float32)
    p2 = (lax.broadcasted_iota(jnp.int32, (HP, D), 1) % HP
          == lax.broadcasted_iota(jnp.int32, (HP, D), 0)).astype(jnp.float32)
    blk = (lax.broadcasted_iota(jnp.int32, (32, D), 0) // 4
           == lax.broadcasted_iota(jnp.int32, (32, D), 1) // HP)
    k = jnp.dot(jnp.dot(p1, m, preferred_element_type=jnp.float32), p2,
                preferred_element_type=jnp.float32) * blk.astype(jnp.float32)
    out_ref[...] = jnp.dot(eattr_ref[...], k,
                           preferred_element_type=jnp.float32)


def _tc_edge(eattr2d, we, aef):
    return pl.pallas_call(
        _edge_body,
        grid=(E_PAD // 8 // EBLK,),
        in_specs=[
            pl.BlockSpec((EBLK, 32), lambda i: (i, 0)),
            pl.BlockSpec((4, D), lambda i: (0, 0)),
            pl.BlockSpec((1, D), lambda i: (0, 0)),
        ],
        out_specs=pl.BlockSpec((EBLK, D), lambda i: (i, 0)),
        out_shape=jax.ShapeDtypeStruct((E_PAD // 8, D), jnp.float32),
    )(eattr2d, we, aef)


def _post_body(acc_ref, ss_ref, x_ref, b_ref, g_ref, be_ref, out_ref):
    acc = acc_ref[0] + acc_ref[1]
    ss = ss_ref[0] + ss_ref[1]
    expand = _sel8(D, HP).T
    den = jnp.dot(ss, expand, preferred_element_type=jnp.float32) + 1e-16
    out = acc / den + b_ref[...] + x_ref[...]
    mu = jnp.mean(out, axis=-1, keepdims=True)
    var = jnp.mean((out - mu) * (out - mu), axis=-1, keepdims=True)
    y = (out - mu) * lax.rsqrt(var + 1e-5) * g_ref[...] + be_ref[...]
    out_ref[...] = jnp.where(y > 0, y, jnp.exp(y) - 1.0)


def _tc_post(acc, ss, x, b, g, be):
    return pl.pallas_call(
        _post_body,
        grid=(N // 1000,),
        in_specs=[
            pl.BlockSpec((NC, 1000, D), lambda i: (0, i, 0)),
            pl.BlockSpec((NC, 1000, HP), lambda i: (0, i, 0)),
            pl.BlockSpec((1000, D), lambda i: (i, 0)),
            pl.BlockSpec((1, D), lambda i: (0, 0)),
            pl.BlockSpec((1, D), lambda i: (0, 0)),
            pl.BlockSpec((1, D), lambda i: (0, 0)),
        ],
        out_specs=pl.BlockSpec((1000, D), lambda i: (i, 0)),
        out_shape=jax.ShapeDtypeStruct((N, D), jnp.float32),
    )(acc, ss, x, b, g, be)


@functools.partial(
    pl.kernel,
    out_type=[
        jax.ShapeDtypeStruct((NC * NPAD, D), jnp.float32),
        jax.ShapeDtypeStruct((NC * NPAD, HP), jnp.float32),
    ],
    mesh=plsc.VectorSubcoreMesh(core_axis_name="c", subcore_axis_name="s"),
    compiler_params=pltpu.CompilerParams(use_tc_tiling_on_sc=False),
    scratch_types=[
        pltpu.VMEM((CHUNK,), jnp.int32),
        pltpu.VMEM((CHUNK,), jnp.int32),
        pltpu.VMEM((CHUNK // 8, D), jnp.float32),
        pltpu.VMEM((CHUNK, HP), jnp.float32),
        pltpu.VMEM((CHUNK, HP), jnp.float32),
        pltpu.VMEM((CHUNK, HP), jnp.float32),
        pltpu.VMEM((CHUNK, D), jnp.float32),
        pltpu.VMEM_SHARED((NPAD, D), jnp.float32),
        pltpu.VMEM_SHARED((NPAD, HP), jnp.float32),
        pltpu.SemaphoreType.DMA,
        pltpu.SemaphoreType.DMA,
        pltpu.SemaphoreType.DMA,
    ],
)
def _sc_edge(xp_hbm, asrc_hbm, adst_hbm, ae_hbm, src_hbm, dst_hbm,
             acc_out, ss_out,
             src_v, dst_v, ae_v, ea_v, as_v, ad_v, x_v,
             acc_sh, ss_sh, sem1, sem2, sem3):
    cid = lax.axis_index("c")
    sid = lax.axis_index("s")
    wid = sid * NC + cid
    row0 = sid * ROWS_PER_TILE

    z16 = jnp.zeros((16,), jnp.float32)

    def _zrow(r, _):
        for j in range(D // 16):
            x_v[r, pl.ds(j * 16, 16)] = z16
        ea_v[r, :] = z16
        return 0

    lax.fori_loop(0, CHUNK, _zrow, 0)
    for k in range(ROWS_PER_TILE // CHUNK):
        pltpu.sync_copy(x_v, acc_sh.at[pl.ds(row0 + k * CHUNK, CHUNK), :])
        pltpu.sync_copy(ea_v, ss_sh.at[pl.ds(row0 + k * CHUNK, CHUNK), :])
    plsc.subcore_barrier()

    def _chunk(t, _):
        c = wid * CHUNKS_PER_W + t
        base = c * CHUNK
        pltpu.sync_copy(src_hbm.at[pl.ds(base, CHUNK)], src_v)
        pltpu.sync_copy(dst_hbm.at[pl.ds(base, CHUNK)], dst_v)
        cp1 = pltpu.async_copy(asrc_hbm.at[src_v], as_v, sem1)
        cp2 = pltpu.async_copy(adst_hbm.at[dst_v], ad_v, sem2)
        cp3 = pltpu.async_copy(xp_hbm.at[src_v], x_v, sem3)
        pltpu.sync_copy(ae_hbm.at[pl.ds(c * (CHUNK // 8), CHUNK // 8), :],
                        ae_v)
        cp1.wait()
        cp2.wait()

        def _erow(r, _):
            for j in range(8):
                e = r * 8 + j
                v = (as_v[e, :] + ad_v[e, :] + ae_v[r, pl.ds(j * HP, 16)])
                ea_v[e, :] = jnp.exp(jnp.maximum(v, 0.2 * v))
            return 0

        lax.fori_loop(0, CHUNK // 8, _erow, 0)
        cp3.wait()

        def _mrow(e, _):
            ea = ea_v[e, :]
            for h in range(8):
                w = _splat(ea, h)
                x_v[e, pl.ds(h * 16, 16)] = x_v[e, pl.ds(h * 16, 16)] * w
            return 0

        lax.fori_loop(0, CHUNK, _mrow, 0)
        pltpu.sync_copy(x_v, acc_sh.at[dst_v], add=True)
        pltpu.sync_copy(ea_v, ss_sh.at[dst_v], add=True)
        return 0

    lax.fori_loop(0, CHUNKS_PER_W, _chunk, 0)
    plsc.subcore_barrier()
    pltpu.sync_copy(acc_sh.at[pl.ds(row0, ROWS_PER_TILE), :],
                    acc_out.at[pl.ds(cid * NPAD + row0, ROWS_PER_TILE), :])
    pltpu.sync_copy(ss_sh.at[pl.ds(row0, ROWS_PER_TILE), :],
                    ss_out.at[pl.ds(cid * NPAD + row0, ROWS_PER_TILE), :])


def kernel(x, edge_index, edge_attr, W_gat, att_src, att_dst, W_edge,
           att_edge, bias, gamma, beta):
    src = edge_index[0].astype(jnp.int32)
    dst = edge_index[1].astype(jnp.int32)
    pad_e = E_PAD - E
    srcp = jnp.concatenate([src, jnp.full((pad_e,), N, jnp.int32)])
    dstp = jnp.concatenate([dst, jnp.full((pad_e,), N, jnp.int32)])
    eap = jnp.concatenate(
        [edge_attr, jnp.zeros((pad_e, edge_attr.shape[1]), jnp.float32)])
    xpad = jnp.concatenate([x, jnp.zeros((NPAD - N, D), jnp.float32)])
    asf = att_src.reshape(1, D)
    adf = att_dst.reshape(1, D)
    aef = att_edge.reshape(1, D)
    xp, asrc, adst = _tc_pre(xpad, W_gat, asf, adf)
    ae = _tc_edge(eap.reshape(E_PAD // 8, 32), W_edge, aef)
    acc, ss = _sc_edge(xp, asrc, adst, ae, srcp, dstp)
    return _tc_post(acc.reshape(NC, NPAD, D), ss.reshape(NC, NPAD, HP),
                    x, bias.reshape(1, D), gamma.reshape(1, D),
                    beta.reshape(1, D))
